# baseline (speedup 1.0000x reference)
import functools

import jax
import jax.numpy as jnp
from jax import lax
from jax.experimental import pallas as pl
from jax.experimental.pallas import tpu as pltpu
from jax.experimental.pallas import tpu_sc as plsc

BATCH = 16384
EMB = 32
PACK = 128 // EMB
NC = 2
NS = 16
NW = NC * NS
BPW = BATCH // NW
CHUNK = 128
NCHUNK = BPW // CHUNK

_MESH = plsc.VectorSubcoreMesh(core_axis_name="c", subcore_axis_name="s")


@functools.partial(
    pl.kernel,
    out_type=(
        jax.ShapeDtypeStruct((BATCH, 128), jnp.float32),
        jax.ShapeDtypeStruct((BATCH, 128), jnp.float32),
    ),
    mesh=_MESH,
    scratch_types=[
        pltpu.VMEM((NCHUNK, CHUNK), jnp.int32),
        pltpu.VMEM((NCHUNK, CHUNK), jnp.int32),
        pltpu.VMEM((BPW, 128), jnp.float32),
        pltpu.SemaphoreType.DMA,
        pltpu.SemaphoreType.DMA,
    ],
)
def _sc_gather(uid_hbm, vid_hbm, utab_hbm, vtab_hbm, uw_hbm, vw_hbm,
               uidx_v, vidx_v, rows_v, gsem, osem):
    wid = lax.axis_index("s") * NC + lax.axis_index("c")
    base = wid * BPW
    pltpu.sync_copy(uid_hbm.at[wid], uidx_v)
    pltpu.sync_copy(vid_hbm.at[wid], vidx_v)
    copies = []
    for j in range(NCHUNK):
        copies.append(pltpu.async_copy(
            utab_hbm.at[uidx_v.at[j]], rows_v.at[pl.ds(j * CHUNK, CHUNK)], gsem))
    for c in copies:
        c.wait()
    out_u = pltpu.async_copy(rows_v, uw_hbm.at[pl.ds(base, BPW)], osem)
    out_u.wait()
    copies = []
    for j in range(NCHUNK):
        copies.append(pltpu.async_copy(
            vtab_hbm.at[vidx_v.at[j]], rows_v.at[pl.ds(j * CHUNK, CHUNK)], gsem))
    for c in copies:
        c.wait()
    pltpu.sync_copy(rows_v, vw_hbm.at[pl.ds(base, BPW)])


_ROWS = 2048


def _mlp_body(uw, vw, uo, vo, w1a, w1b, b1, wo1, bo1, wo2, bo2, l1, l2):
    colgrp = lax.broadcasted_iota(jnp.int32, (_ROWS, 128), 1) // EMB
    um = jnp.where(colgrp == uo[...], uw[...], 0.0)
    vm = jnp.where(colgrp == vo[...], vw[...], 0.0)
    h = jnp.dot(um, w1a[...], preferred_element_type=jnp.float32)
    h += jnp.dot(vm, w1b[...], preferred_element_type=jnp.float32)
    h = jnp.maximum(h + b1[...], 0.0)
    l1[...] = jnp.dot(h, wo1[...], preferred_element_type=jnp.float32) + bo1[...]
    l2[...] = jnp.dot(h, wo2[...], preferred_element_type=jnp.float32) + bo2[...]


def _mlp(uw, vw, uo, vo, w1a, w1b, b1, wo1, bo1, wo2, bo2):
    grid = (BATCH // _ROWS,)
    full = lambda shape: pl.BlockSpec(shape, lambda i: (0, 0))
    return pl.pallas_call(
        _mlp_body,
        grid=grid,
        in_specs=[
            pl.BlockSpec((_ROWS, 128), lambda i: (i, 0)),
            pl.BlockSpec((_ROWS, 128), lambda i: (i, 0)),
            pl.BlockSpec((_ROWS, 1), lambda i: (i, 0)),
            pl.BlockSpec((_ROWS, 1), lambda i: (i, 0)),
            full((128, 32)),
            full((128, 32)),
            full((1, 32)),
            full((32, 10)),
            full((1, 10)),
            full((32, 1)),
            full((1, 1)),
        ],
        out_specs=[
            pl.BlockSpec((_ROWS, 10), lambda i: (i, 0)),
            pl.BlockSpec((_ROWS, 1), lambda i: (i, 0)),
        ],
        out_shape=[
            jax.ShapeDtypeStruct((BATCH, 10), jnp.float32),
            jax.ShapeDtypeStruct((BATCH, 1), jnp.float32),
        ],
    )(uw, vw, uo, vo, w1a, w1b, b1, wo1, bo1, wo2, bo2)


def kernel(user_id, video_id, user_table, video_table, W1, b1, Wo1, bo1, Wo2, bo2):
    uid = jnp.asarray(user_id, jnp.int32)
    vid = jnp.asarray(video_id, jnp.int32)
    uwide = jnp.asarray(user_table, jnp.float32).reshape(-1, 128)
    vwide = jnp.asarray(video_table, jnp.float32).reshape(-1, 128)
    uw, vw = _sc_gather(
        (uid // PACK).reshape(NW, NCHUNK, CHUNK),
        (vid // PACK).reshape(NW, NCHUNK, CHUNK),
        uwide, vwide)
    w1rep_a = jnp.tile(W1[:EMB], (PACK, 1))
    w1rep_b = jnp.tile(W1[EMB:], (PACK, 1))
    l1, l2 = _mlp(uw, vw,
                  (uid % PACK).reshape(BATCH, 1), (vid % PACK).reshape(BATCH, 1),
                  w1rep_a, w1rep_b, b1.reshape(1, 32),
                  Wo1, bo1.reshape(1, 10), Wo2, bo2.reshape(1, 1))
    return (l1, l2)


# speedup vs baseline: 2.5136x; 2.5136x over previous
import functools

import jax
import jax.numpy as jnp
from jax import lax
from jax.experimental import pallas as pl
from jax.experimental.pallas import tpu as pltpu
from jax.experimental.pallas import tpu_sc as plsc

BATCH = 16384
EMB = 32
NC = 2
NS = 16
NW = NC * NS
BPW = BATCH // NW
CHUNK = 128
NCHUNK = BPW // CHUNK

S_U, SH_U = 262144, 18
S_V, SH_V = 32768, 15
BS = 2048

_MESH = plsc.VectorSubcoreMesh(core_axis_name="c", subcore_axis_name="s")


def _transposer_body(x0, x1, x2, x3, o):
    x = jnp.concatenate([x0[...], x1[...], x2[...], x3[...]], axis=0)
    r = lax.broadcasted_iota(jnp.int32, (128, 128), 0)
    c = lax.broadcasted_iota(jnp.int32, (128, 128), 1)
    eye = (r == c).astype(jnp.float32)
    dn = (((0,), (0,)), ((), ()))
    o[...] = lax.dot_general(x, eye, dn, preferred_element_type=jnp.float32)


def _widen(table, s):
    n = table.shape[0]
    tab_t = table.T
    k = s // BS
    last = (n + BS - 1) // BS - 1
    specs = [
        pl.BlockSpec(
            (EMB, BS),
            functools.partial(lambda u, i: (0, jnp.minimum(u * k + i, last)), u))
        for u in range(4)
    ]
    return pl.pallas_call(
        _transposer_body,
        grid=(k,),
        in_specs=specs,
        out_specs=pl.BlockSpec((BS, 128), lambda i: (i, 0)),
        out_shape=jax.ShapeDtypeStruct((s, 128), jnp.float32),
    )(tab_t, tab_t, tab_t, tab_t)


@functools.partial(
    pl.kernel,
    out_type=(
        jax.ShapeDtypeStruct((BATCH, 128), jnp.float32),
        jax.ShapeDtypeStruct((BATCH, 128), jnp.float32),
    ),
    mesh=_MESH,
    scratch_types=[
        pltpu.VMEM((NCHUNK, CHUNK), jnp.int32),
        pltpu.VMEM((NCHUNK, CHUNK), jnp.int32),
        pltpu.VMEM((BPW, 128), jnp.float32),
        pltpu.SemaphoreType.DMA,
        pltpu.SemaphoreType.DMA,
    ],
)
def _sc_gather(uid_hbm, vid_hbm, utab_hbm, vtab_hbm, uw_hbm, vw_hbm,
               uidx_v, vidx_v, rows_v, gsem, osem):
    wid = lax.axis_index("s") * NC + lax.axis_index("c")
    base = wid * BPW
    pltpu.sync_copy(uid_hbm.at[wid], uidx_v)
    pltpu.sync_copy(vid_hbm.at[wid], vidx_v)
    copies = []
    for j in range(NCHUNK):
        copies.append(pltpu.async_copy(
            utab_hbm.at[uidx_v.at[j]], rows_v.at[pl.ds(j * CHUNK, CHUNK)], gsem))
    for c in copies:
        c.wait()
    out_u = pltpu.async_copy(rows_v, uw_hbm.at[pl.ds(base, BPW)], osem)
    out_u.wait()
    copies = []
    for j in range(NCHUNK):
        copies.append(pltpu.async_copy(
            vtab_hbm.at[vidx_v.at[j]], rows_v.at[pl.ds(j * CHUNK, CHUNK)], gsem))
    for c in copies:
        c.wait()
    pltpu.sync_copy(rows_v, vw_hbm.at[pl.ds(base, BPW)])


_ROWS = 2048


def _mlp_body(uw, vw, uo, vo, w1a, w1b, b1, wo1, bo1, wo2, bo2, l1, l2):
    colgrp = lax.broadcasted_iota(jnp.int32, (_ROWS, 128), 1) // EMB
    um = jnp.where(colgrp == uo[...], uw[...], 0.0)
    vm = jnp.where(colgrp == vo[...], vw[...], 0.0)
    h = jnp.dot(um, w1a[...], preferred_element_type=jnp.float32)
    h += jnp.dot(vm, w1b[...], preferred_element_type=jnp.float32)
    h = jnp.maximum(h + b1[...], 0.0)
    l1[...] = jnp.dot(h, wo1[...], preferred_element_type=jnp.float32) + bo1[...]
    l2[...] = jnp.dot(h, wo2[...], preferred_element_type=jnp.float32) + bo2[...]


def _mlp(uw, vw, uo, vo, w1a, w1b, b1, wo1, bo1, wo2, bo2):
    grid = (BATCH // _ROWS,)
    full = lambda shape: pl.BlockSpec(shape, lambda i: (0, 0))
    return pl.pallas_call(
        _mlp_body,
        grid=grid,
        in_specs=[
            pl.BlockSpec((_ROWS, 128), lambda i: (i, 0)),
            pl.BlockSpec((_ROWS, 128), lambda i: (i, 0)),
            pl.BlockSpec((_ROWS, 1), lambda i: (i, 0)),
            pl.BlockSpec((_ROWS, 1), lambda i: (i, 0)),
            full((128, 32)),
            full((128, 32)),
            full((1, 32)),
            full((32, 10)),
            full((1, 10)),
            full((32, 1)),
            full((1, 1)),
        ],
        out_specs=[
            pl.BlockSpec((_ROWS, 10), lambda i: (i, 0)),
            pl.BlockSpec((_ROWS, 1), lambda i: (i, 0)),
        ],
        out_shape=[
            jax.ShapeDtypeStruct((BATCH, 10), jnp.float32),
            jax.ShapeDtypeStruct((BATCH, 1), jnp.float32),
        ],
    )(uw, vw, uo, vo, w1a, w1b, b1, wo1, bo1, wo2, bo2)


def kernel(user_id, video_id, user_table, video_table, W1, b1, Wo1, bo1, Wo2, bo2):
    uid = jnp.asarray(user_id, jnp.int32)
    vid = jnp.asarray(video_id, jnp.int32)
    uwide = _widen(user_table, S_U)
    vwide = _widen(video_table, S_V)
    uw, vw = _sc_gather(
        (uid & (S_U - 1)).reshape(NW, NCHUNK, CHUNK),
        (vid & (S_V - 1)).reshape(NW, NCHUNK, CHUNK),
        uwide, vwide)
    w1rep_a = jnp.tile(W1[:EMB], (4, 1))
    w1rep_b = jnp.tile(W1[EMB:], (4, 1))
    l1, l2 = _mlp(uw, vw,
                  (uid >> SH_U).reshape(BATCH, 1), (vid >> SH_V).reshape(BATCH, 1),
                  w1rep_a, w1rep_b, b1.reshape(1, 32),
                  Wo1, bo1.reshape(1, 10), Wo2, bo2.reshape(1, 1))
    return (l1, l2)


# speedup vs baseline: 3.4248x; 1.3625x over previous
import functools

import jax
import jax.numpy as jnp
from jax import lax
from jax.experimental import pallas as pl
from jax.experimental.pallas import tpu as pltpu
from jax.experimental.pallas import tpu_sc as plsc

BATCH = 16384
EMB = 32
NC = 2
NS = 16
NW = NC * NS
BPW = BATCH // NW
CHUNK = 128
NCHUNK = BPW // CHUNK

S_U, SH_U = 262144, 18
S_V, SH_V = 32768, 15
BS = 4096

_MESH = plsc.VectorSubcoreMesh(core_axis_name="c", subcore_axis_name="s")


def _transposer_body(x0, x1, x2, x3, o):
    x = jnp.concatenate([x0[...], x1[...], x2[...], x3[...]], axis=0)
    r = lax.broadcasted_iota(jnp.int32, (128, 128), 0)
    c = lax.broadcasted_iota(jnp.int32, (128, 128), 1)
    eye = (r == c).astype(jnp.float32)
    dn = (((0,), (0,)), ((), ()))
    o[...] = lax.dot_general(x, eye, dn, preferred_element_type=jnp.float32)


def _widen(table, s):
    n = table.shape[0]
    tab_t = table.T
    k = s // BS
    last = (n + BS - 1) // BS - 1
    specs = [
        pl.BlockSpec(
            (EMB, BS),
            functools.partial(lambda u, i: (0, jnp.minimum(u * k + i, last)), u))
        for u in range(4)
    ]
    return pl.pallas_call(
        _transposer_body,
        grid=(k,),
        in_specs=specs,
        out_specs=pl.BlockSpec((BS, 128), lambda i: (i, 0)),
        out_shape=jax.ShapeDtypeStruct((s, 128), jnp.float32),
    )(tab_t, tab_t, tab_t, tab_t)


@functools.partial(
    pl.kernel,
    out_type=(
        jax.ShapeDtypeStruct((BATCH, 128), jnp.float32),
        jax.ShapeDtypeStruct((BATCH, 128), jnp.float32),
    ),
    mesh=_MESH,
    scratch_types=[
        pltpu.VMEM((NCHUNK, CHUNK), jnp.int32),
        pltpu.VMEM((NCHUNK, CHUNK), jnp.int32),
        pltpu.VMEM((BPW, 128), jnp.float32),
        pltpu.SemaphoreType.DMA,
        pltpu.SemaphoreType.DMA,
    ],
)
def _sc_gather(uid_hbm, vid_hbm, utab_hbm, vtab_hbm, uw_hbm, vw_hbm,
               uidx_v, vidx_v, rows_v, gsem, osem):
    wid = lax.axis_index("s") * NC + lax.axis_index("c")
    base = wid * BPW
    pltpu.sync_copy(uid_hbm.at[wid], uidx_v)
    pltpu.sync_copy(vid_hbm.at[wid], vidx_v)
    copies = []
    for j in range(NCHUNK):
        copies.append(pltpu.async_copy(
            utab_hbm.at[uidx_v.at[j]], rows_v.at[pl.ds(j * CHUNK, CHUNK)], gsem))
    for c in copies:
        c.wait()
    out_u = pltpu.async_copy(rows_v, uw_hbm.at[pl.ds(base, BPW)], osem)
    out_u.wait()
    copies = []
    for j in range(NCHUNK):
        copies.append(pltpu.async_copy(
            vtab_hbm.at[vidx_v.at[j]], rows_v.at[pl.ds(j * CHUNK, CHUNK)], gsem))
    for c in copies:
        c.wait()
    pltpu.sync_copy(rows_v, vw_hbm.at[pl.ds(base, BPW)])


_ROWS = 2048


def _mlp_body(uw, vw, uo, vo, w1a, w1b, b1, wo1, bo1, wo2, bo2, l1, l2):
    colgrp = lax.broadcasted_iota(jnp.int32, (_ROWS, 128), 1) // EMB
    um = jnp.where(colgrp == uo[...], uw[...], 0.0)
    vm = jnp.where(colgrp == vo[...], vw[...], 0.0)
    h = jnp.dot(um, w1a[...], preferred_element_type=jnp.float32)
    h += jnp.dot(vm, w1b[...], preferred_element_type=jnp.float32)
    h = jnp.maximum(h + b1[...], 0.0)
    dn = (((0,), (1,)), ((), ()))
    l1[...] = lax.dot_general(wo1[...], h, dn,
                              preferred_element_type=jnp.float32) + bo1[...]
    l2[...] = lax.dot_general(wo2[...], h, dn,
                              preferred_element_type=jnp.float32) + bo2[...]


def _mlp(uw, vw, uo, vo, w1a, w1b, b1, wo1, bo1, wo2, bo2):
    grid = (BATCH // _ROWS,)
    full = lambda shape: pl.BlockSpec(shape, lambda i: (0, 0))
    return pl.pallas_call(
        _mlp_body,
        grid=grid,
        in_specs=[
            pl.BlockSpec((_ROWS, 128), lambda i: (i, 0)),
            pl.BlockSpec((_ROWS, 128), lambda i: (i, 0)),
            pl.BlockSpec((_ROWS, 1), lambda i: (i, 0)),
            pl.BlockSpec((_ROWS, 1), lambda i: (i, 0)),
            full((128, 32)),
            full((128, 32)),
            full((1, 32)),
            full((32, 10)),
            full((10, 1)),
            full((32, 1)),
            full((1, 1)),
        ],
        out_specs=[
            pl.BlockSpec((10, _ROWS), lambda i: (0, i)),
            pl.BlockSpec((1, _ROWS), lambda i: (0, i)),
        ],
        out_shape=[
            jax.ShapeDtypeStruct((10, BATCH), jnp.float32),
            jax.ShapeDtypeStruct((1, BATCH), jnp.float32),
        ],
    )(uw, vw, uo, vo, w1a, w1b, b1, wo1, bo1, wo2, bo2)


def kernel(user_id, video_id, user_table, video_table, W1, b1, Wo1, bo1, Wo2, bo2):
    uid = jnp.asarray(user_id, jnp.int32)
    vid = jnp.asarray(video_id, jnp.int32)
    uwide = _widen(user_table, S_U)
    vwide = _widen(video_table, S_V)
    uw, vw = _sc_gather(
        (uid & (S_U - 1)).reshape(NW, NCHUNK, CHUNK),
        (vid & (S_V - 1)).reshape(NW, NCHUNK, CHUNK),
        uwide, vwide)
    w1rep_a = jnp.tile(W1[:EMB], (4, 1))
    w1rep_b = jnp.tile(W1[EMB:], (4, 1))
    l1t, l2t = _mlp(uw, vw,
                    (uid >> SH_U).reshape(BATCH, 1), (vid >> SH_V).reshape(BATCH, 1),
                    w1rep_a, w1rep_b, b1.reshape(1, 32),
                    Wo1, bo1.reshape(10, 1), Wo2, bo2.reshape(1, 1))
    return (l1t.T, l2t.T)
